# baseline (device time: 69272 ns/iter reference)
import jax
import jax.numpy as jnp
from jax import lax
from jax.experimental import pallas as pl
from jax.experimental.pallas import tpu as pltpu

N_DEV = 4
SCALE = 0.08838834764831843
GQA_REP = 4


def kernel(x, Wq, Wo, K_ext, V_ext):
    B, Sq, D = x.shape
    Dq = Wq.shape[1]
    Dh = K_ext.shape[-1]
    n_local_heads = Dq // Dh
    n_local_kv = n_local_heads // GQA_REP

    my_i = lax.axis_index("i")
    xs = x[0]
    Kl = lax.dynamic_slice_in_dim(K_ext[0], n_local_kv * my_i, n_local_kv, axis=1)
    Vl = lax.dynamic_slice_in_dim(V_ext[0], n_local_kv * my_i, n_local_kv, axis=1)
    Kl = Kl.transpose(1, 0, 2)
    Vl = Vl.transpose(1, 0, 2)

    def body(x_ref, wq_ref, wo_ref, k_ref, v_ref, out_ref,
             stage_p, stage_m, send_p, recv_p, send_m, recv_m):
        my_pos = lax.axis_index("i")
        left = lax.rem(my_pos + N_DEV - 1, N_DEV)
        right = lax.rem(my_pos + 1, N_DEV)

        barrier_sem = pltpu.get_barrier_semaphore()
        for nbr in (left, right):
            pl.semaphore_signal(barrier_sem, inc=1, device_id=(nbr,),
                                device_id_type=pl.DeviceIdType.MESH)
        pl.semaphore_wait(barrier_sem, 2)

        CH = Sq // (2 * N_DEV)
        HB = Sq // 2

        def compute_chunk(row_start):
            x_c = x_ref[pl.ds(row_start, CH), :]
            q_c = jnp.dot(x_c, wq_ref[:, :],
                          preferred_element_type=jnp.float32)
            outs = []
            for h in range(n_local_heads):
                g = h // GQA_REP
                q_h = q_c[:, h * Dh:(h + 1) * Dh]
                s = lax.dot_general(q_h, k_ref[g], (((1,), (1,)), ((), ())),
                                    preferred_element_type=jnp.float32) * SCALE
                m = jnp.max(s, axis=1, keepdims=True)
                pj = jnp.exp(s - m)
                l = jnp.sum(pj, axis=1, keepdims=True)
                o = jnp.dot(pj, v_ref[g],
                            preferred_element_type=jnp.float32) / l
                outs.append(o)
            attn = jnp.concatenate(outs, axis=1)
            return jnp.dot(attn, wo_ref[:, :],
                           preferred_element_type=jnp.float32)

        p = my_pos

        def mod4(v):
            return lax.rem(v + 2 * N_DEV, N_DEV)

        st0 = CH * mod4(p)
        sb0 = HB + CH * mod4(p)
        out_ref[pl.ds(st0, CH), :] = compute_chunk(st0)
        out_ref[pl.ds(sb0, CH), :] = compute_chunk(sb0)

        for s in range(N_DEV - 1):
            cs_p, cr_p = mod4(p - s), mod4(p - s - 1)
            cs_m, cr_m = mod4(p + s), mod4(p + s + 1)
            r_p = pltpu.make_async_remote_copy(
                src_ref=out_ref.at[pl.ds(CH * cs_p, CH), :],
                dst_ref=stage_p.at[s],
                send_sem=send_p.at[s], recv_sem=recv_p.at[s],
                device_id=(right,), device_id_type=pl.DeviceIdType.MESH,
            )
            r_m = pltpu.make_async_remote_copy(
                src_ref=out_ref.at[pl.ds(HB + CH * cs_m, CH), :],
                dst_ref=stage_m.at[s],
                send_sem=send_m.at[s], recv_sem=recv_m.at[s],
                device_id=(left,), device_id_type=pl.DeviceIdType.MESH,
            )
            r_p.start()
            r_m.start()
            ct = CH * cr_p
            cb = HB + CH * cr_m
            out_ref[pl.ds(ct, CH), :] = compute_chunk(ct)
            out_ref[pl.ds(cb, CH), :] = compute_chunk(cb)
            r_p.wait()
            r_m.wait()
            out_ref[pl.ds(ct, CH), :] += stage_p[s]
            out_ref[pl.ds(cb, CH), :] += stage_m[s]

        for s in range(N_DEV - 1):
            k = N_DEV - 1 + s
            cs_p, cr_p = mod4(p + 1 - s), mod4(p - s)
            cs_m, cr_m = mod4(p - 1 + s), mod4(p + s)
            r_p = pltpu.make_async_remote_copy(
                src_ref=out_ref.at[pl.ds(CH * cs_p, CH), :],
                dst_ref=out_ref.at[pl.ds(CH * cs_p, CH), :],
                send_sem=send_p.at[k], recv_sem=recv_p.at[k],
                device_id=(right,), device_id_type=pl.DeviceIdType.MESH,
            )
            r_m = pltpu.make_async_remote_copy(
                src_ref=out_ref.at[pl.ds(HB + CH * cs_m, CH), :],
                dst_ref=out_ref.at[pl.ds(HB + CH * cs_m, CH), :],
                send_sem=send_m.at[k], recv_sem=recv_m.at[k],
                device_id=(left,), device_id_type=pl.DeviceIdType.MESH,
            )
            r_p.start()
            r_m.start()
            r_p.wait()
            r_m.wait()

    out = pl.pallas_call(
        body,
        out_shape=jax.ShapeDtypeStruct((Sq, D), jnp.float32),
        in_specs=[pl.BlockSpec(memory_space=pltpu.VMEM)] * 5,
        out_specs=pl.BlockSpec(memory_space=pltpu.VMEM),
        scratch_shapes=[
            pltpu.VMEM((N_DEV - 1, Sq // (2 * N_DEV), D), jnp.float32),
            pltpu.VMEM((N_DEV - 1, Sq // (2 * N_DEV), D), jnp.float32),
            pltpu.SemaphoreType.DMA((2 * (N_DEV - 1),)),
            pltpu.SemaphoreType.DMA((2 * (N_DEV - 1),)),
            pltpu.SemaphoreType.DMA((2 * (N_DEV - 1),)),
            pltpu.SemaphoreType.DMA((2 * (N_DEV - 1),)),
        ],
        compiler_params=pltpu.CompilerParams(collective_id=0),
    )(xs, Wq, Wo, Kl, Vl)
    return out.reshape(B, Sq, D)


# device time: 48985 ns/iter; 1.4141x vs baseline; 1.4141x over previous
import jax
import jax.numpy as jnp
from jax import lax
from jax.experimental import pallas as pl
from jax.experimental.pallas import tpu as pltpu

N_DEV = 4
SCALE = 0.08838834764831843
GQA_REP = 4


def kernel(x, Wq, Wo, K_ext, V_ext):
    B, Sq, D = x.shape
    Dq = Wq.shape[1]
    Dh = K_ext.shape[-1]
    n_local_heads = Dq // Dh
    n_local_kv = n_local_heads // GQA_REP
    CH = Sq // (2 * N_DEV)
    HB = Sq // 2

    my_i = lax.axis_index("i")
    xs = x[0]
    Kl = lax.dynamic_slice_in_dim(K_ext[0], n_local_kv * my_i, n_local_kv, axis=1)
    Vl = lax.dynamic_slice_in_dim(V_ext[0], n_local_kv * my_i, n_local_kv, axis=1)
    Kl = Kl.transpose(1, 0, 2)
    Vl = Vl.transpose(1, 0, 2)

    def body(x_ref, wq_ref, wo_ref, k_ref, v_ref, out_ref,
             stage_p, stage_m, sbuf_p, sbuf_m, send_p, recv_p, send_m, recv_m):
        p = lax.axis_index("i")
        left = lax.rem(p + N_DEV - 1, N_DEV)
        right = lax.rem(p + 1, N_DEV)

        def mod4(v):
            return lax.rem(v + 2 * N_DEV, N_DEV)

        barrier_sem = pltpu.get_barrier_semaphore()
        for nbr in (left, right):
            pl.semaphore_signal(barrier_sem, inc=1, device_id=(nbr,),
                                device_id_type=pl.DeviceIdType.MESH)
        pl.semaphore_wait(barrier_sem, 2)

        xb = x_ref[:, :].astype(jnp.bfloat16)
        wqb = wq_ref[:, :].astype(jnp.bfloat16)
        q = jnp.dot(xb, wqb, preferred_element_type=jnp.float32)
        outs = []
        for h in range(n_local_heads):
            g = h // GQA_REP
            q_h = q[:, h * Dh:(h + 1) * Dh].astype(jnp.bfloat16)
            kb = k_ref[g].astype(jnp.bfloat16)
            s = lax.dot_general(q_h, kb, (((1,), (1,)), ((), ())),
                                preferred_element_type=jnp.float32) * SCALE
            m = jnp.max(s, axis=1, keepdims=True)
            pj = jnp.exp(s - m)
            l = jnp.sum(pj, axis=1, keepdims=True)
            o = jnp.dot(pj.astype(jnp.bfloat16), v_ref[g].astype(jnp.bfloat16),
                        preferred_element_type=jnp.float32) / l
            outs.append(o)
        attn = jnp.concatenate(outs, axis=1).astype(jnp.bfloat16)
        partial = jnp.dot(attn, wo_ref[:, :].astype(jnp.bfloat16),
                          preferred_element_type=jnp.float32)
        out_ref[:, :] = partial


        for s in range(N_DEV - 1):
            cs_p, cr_p = mod4(p - s), mod4(p - s - 1)
            cs_m, cr_m = mod4(p + s), mod4(p + s + 1)
            sbuf_p[s] = out_ref[pl.ds(CH * cs_p, CH), :].astype(jnp.bfloat16)
            sbuf_m[s] = out_ref[pl.ds(HB + CH * cs_m, CH), :].astype(jnp.bfloat16)
            r_p = pltpu.make_async_remote_copy(
                src_ref=sbuf_p.at[s],
                dst_ref=stage_p.at[s],
                send_sem=send_p.at[s], recv_sem=recv_p.at[s],
                device_id=(right,), device_id_type=pl.DeviceIdType.MESH,
            )
            r_m = pltpu.make_async_remote_copy(
                src_ref=sbuf_m.at[s],
                dst_ref=stage_m.at[s],
                send_sem=send_m.at[s], recv_sem=recv_m.at[s],
                device_id=(left,), device_id_type=pl.DeviceIdType.MESH,
            )
            r_p.start()
            r_m.start()
            r_p.wait()
            r_m.wait()
            out_ref[pl.ds(CH * cr_p, CH), :] += stage_p[s].astype(jnp.float32)
            out_ref[pl.ds(HB + CH * cr_m, CH), :] += stage_m[s].astype(jnp.float32)

        sbuf_p[N_DEV - 1] = out_ref[pl.ds(CH * mod4(p + 1), CH), :].astype(jnp.bfloat16)
        sbuf_m[N_DEV - 1] = out_ref[pl.ds(HB + CH * mod4(p - 1), CH), :].astype(jnp.bfloat16)
        for s in range(N_DEV - 1):
            k = N_DEV - 1 + s
            cr_p = mod4(p - s)
            cr_m = mod4(p + s)
            src_p = sbuf_p.at[N_DEV - 1] if s == 0 else stage_p.at[k - 1]
            src_m = sbuf_m.at[N_DEV - 1] if s == 0 else stage_m.at[k - 1]
            r_p = pltpu.make_async_remote_copy(
                src_ref=src_p,
                dst_ref=stage_p.at[k],
                send_sem=send_p.at[k], recv_sem=recv_p.at[k],
                device_id=(right,), device_id_type=pl.DeviceIdType.MESH,
            )
            r_m = pltpu.make_async_remote_copy(
                src_ref=src_m,
                dst_ref=stage_m.at[k],
                send_sem=send_m.at[k], recv_sem=recv_m.at[k],
                device_id=(left,), device_id_type=pl.DeviceIdType.MESH,
            )
            r_p.start()
            r_m.start()
            r_p.wait()
            r_m.wait()
            out_ref[pl.ds(CH * cr_p, CH), :] = stage_p[k].astype(jnp.float32)
            out_ref[pl.ds(HB + CH * cr_m, CH), :] = stage_m[k].astype(jnp.float32)

    n_sem = 2 * (N_DEV - 1)
    out = pl.pallas_call(
        body,
        out_shape=jax.ShapeDtypeStruct((Sq, D), jnp.float32),
        in_specs=[pl.BlockSpec(memory_space=pltpu.VMEM)] * 5,
        out_specs=pl.BlockSpec(memory_space=pltpu.VMEM),
        scratch_shapes=[
            pltpu.VMEM((n_sem, CH, D), jnp.bfloat16),
            pltpu.VMEM((n_sem, CH, D), jnp.bfloat16),
            pltpu.VMEM((N_DEV, CH, D), jnp.bfloat16),
            pltpu.VMEM((N_DEV, CH, D), jnp.bfloat16),
            pltpu.SemaphoreType.DMA((n_sem,)),
            pltpu.SemaphoreType.DMA((n_sem,)),
            pltpu.SemaphoreType.DMA((n_sem,)),
            pltpu.SemaphoreType.DMA((n_sem,)),
        ],
        compiler_params=pltpu.CompilerParams(collective_id=0),
    )(xs, Wq, Wo, Kl, Vl)
    return out.reshape(B, Sq, D)


# device time: 47296 ns/iter; 1.4646x vs baseline; 1.0357x over previous
import jax
import jax.numpy as jnp
from jax import lax
from jax.experimental import pallas as pl
from jax.experimental.pallas import tpu as pltpu

N_DEV = 4
SCALE = 0.08838834764831843
GQA_REP = 4


def kernel(x, Wq, Wo, K_ext, V_ext):
    B, Sq, D = x.shape
    Dq = Wq.shape[1]
    Dh = K_ext.shape[-1]
    Skv = K_ext.shape[1]
    n_local_heads = Dq // Dh
    n_local_kv = n_local_heads // GQA_REP
    CH = Sq // N_DEV

    my_i = lax.axis_index("i")
    xs = x[0]
    Kl = lax.dynamic_slice_in_dim(K_ext[0], n_local_kv * my_i, n_local_kv,
                                  axis=1).reshape(Skv, n_local_kv * Dh)
    Vl = lax.dynamic_slice_in_dim(V_ext[0], n_local_kv * my_i, n_local_kv,
                                  axis=1).reshape(Skv, n_local_kv * Dh)

    def body(x_ref, wq_ref, wo_ref, k_ref, v_ref, out_ref,
             wqb_ref, wob_ref, kb_ref, vb_ref,
             sbuf_rs, stage_rs, sbuf_ag, stage_ag,
             rs_send, rs_recv, ag_send, ag_recv, dummy_sem):
        p = lax.axis_index("i")

        def mod4(v):
            return lax.rem(v + 2 * N_DEV, N_DEV)

        barrier_sem = pltpu.get_barrier_semaphore()
        for delta in (1, 2, 3):
            pl.semaphore_signal(barrier_sem, inc=1, device_id=(mod4(p + delta),),
                                device_id_type=pl.DeviceIdType.MESH)
        pl.semaphore_wait(barrier_sem, 3)

        wqb_ref[:, :] = wq_ref[:, :].astype(jnp.bfloat16)
        wob_ref[:, :] = wo_ref[:, :].astype(jnp.bfloat16)
        kb_ref[:, :] = k_ref[:, :].astype(jnp.bfloat16)
        vb_ref[:, :] = v_ref[:, :].astype(jnp.bfloat16)

        def compute_chunk(row):
            xb = x_ref[pl.ds(row, CH), :].astype(jnp.bfloat16)
            q_c = jnp.dot(xb, wqb_ref[:, :],
                          preferred_element_type=jnp.float32)
            outs = []
            for h in range(n_local_heads):
                g = h // GQA_REP
                q_h = q_c[:, h * Dh:(h + 1) * Dh].astype(jnp.bfloat16)
                s = lax.dot_general(
                    q_h, kb_ref[:, g * Dh:(g + 1) * Dh],
                    (((1,), (1,)), ((), ())),
                    preferred_element_type=jnp.float32) * SCALE
                m = jnp.max(s, axis=1, keepdims=True)
                pj = jnp.exp(s - m)
                l = jnp.sum(pj, axis=1, keepdims=True)
                o = jnp.dot(pj.astype(jnp.bfloat16),
                            vb_ref[:, g * Dh:(g + 1) * Dh],
                            preferred_element_type=jnp.float32) / l
                outs.append(o)
            attn = jnp.concatenate(outs, axis=1).astype(jnp.bfloat16)
            return jnp.dot(attn, wob_ref[:, :],
                           preferred_element_type=jnp.float32)

        for delta in (2, 1, 3):
            o = mod4(p + delta)
            part = compute_chunk(CH * o)
            myslot, dstslot = delta - 1, 3 - delta
            sbuf_rs[myslot] = part.astype(jnp.bfloat16)
            rdma = pltpu.make_async_remote_copy(
                src_ref=sbuf_rs.at[myslot],
                dst_ref=stage_rs.at[dstslot],
                send_sem=rs_send.at[myslot], recv_sem=rs_recv.at[dstslot],
                device_id=(o,), device_id_type=pl.DeviceIdType.MESH,
            )
            rdma.start()

        own = compute_chunk(CH * p)

        for j in range(3):
            rr = pltpu.make_async_remote_copy(
                src_ref=stage_rs.at[j], dst_ref=stage_rs.at[j],
                send_sem=dummy_sem.at[j], recv_sem=rs_recv.at[j],
                device_id=(p,), device_id_type=pl.DeviceIdType.MESH,
            )
            rr.wait_recv()

        red = (own + stage_rs[0].astype(jnp.float32)
               + stage_rs[1].astype(jnp.float32)
               + stage_rs[2].astype(jnp.float32))
        out_ref[pl.ds(CH * p, CH), :] = red
        sbuf_ag[:, :] = red.astype(jnp.bfloat16)

        for delta in (1, 2, 3):
            rdma = pltpu.make_async_remote_copy(
                src_ref=sbuf_ag,
                dst_ref=stage_ag.at[3 - delta],
                send_sem=ag_send.at[delta - 1], recv_sem=ag_recv.at[3 - delta],
                device_id=(mod4(p + delta),),
                device_id_type=pl.DeviceIdType.MESH,
            )
            rdma.start()

        for j in range(3):
            ra = pltpu.make_async_remote_copy(
                src_ref=stage_ag.at[j], dst_ref=stage_ag.at[j],
                send_sem=dummy_sem.at[j], recv_sem=ag_recv.at[j],
                device_id=(p,), device_id_type=pl.DeviceIdType.MESH,
            )
            ra.wait_recv()
            src_owner = mod4(p + j + 1)
            out_ref[pl.ds(CH * src_owner, CH), :] = stage_ag[j].astype(jnp.float32)

        for j in range(3):
            ws = pltpu.make_async_remote_copy(
                src_ref=sbuf_rs.at[j], dst_ref=stage_rs.at[j],
                send_sem=rs_send.at[j], recv_sem=dummy_sem.at[j],
                device_id=(p,), device_id_type=pl.DeviceIdType.MESH,
            )
            ws.wait_send()
            wa = pltpu.make_async_remote_copy(
                src_ref=sbuf_ag, dst_ref=stage_ag.at[j],
                send_sem=ag_send.at[j], recv_sem=dummy_sem.at[j],
                device_id=(p,), device_id_type=pl.DeviceIdType.MESH,
            )
            wa.wait_send()

    out = pl.pallas_call(
        body,
        out_shape=jax.ShapeDtypeStruct((Sq, D), jnp.float32),
        in_specs=[pl.BlockSpec(memory_space=pltpu.VMEM)] * 5,
        out_specs=pl.BlockSpec(memory_space=pltpu.VMEM),
        scratch_shapes=[
            pltpu.VMEM((Wq.shape[0], Dq), jnp.bfloat16),
            pltpu.VMEM((Dq, D), jnp.bfloat16),
            pltpu.VMEM((Skv, n_local_kv * Dh), jnp.bfloat16),
            pltpu.VMEM((Skv, n_local_kv * Dh), jnp.bfloat16),
            pltpu.VMEM((3, CH, D), jnp.bfloat16),
            pltpu.VMEM((3, CH, D), jnp.bfloat16),
            pltpu.VMEM((CH, D), jnp.bfloat16),
            pltpu.VMEM((3, CH, D), jnp.bfloat16),
            pltpu.SemaphoreType.DMA((3,)),
            pltpu.SemaphoreType.DMA((3,)),
            pltpu.SemaphoreType.DMA((3,)),
            pltpu.SemaphoreType.DMA((3,)),
            pltpu.SemaphoreType.DMA((3,)),
        ],
        compiler_params=pltpu.CompilerParams(collective_id=0),
    )(xs, Wq, Wo, Kl, Vl)
    return out.reshape(B, Sq, D)
